# Initial kernel scaffold; baseline (speedup 1.0000x reference)
import jax
import jax.numpy as jnp
from jax.experimental import pallas as pl


def kernel(inp_seq, trg_seq, h0, c0, emb_W, emb_b, lstm_W_ih, lstm_W_hh, lstm_b_ih, lstm_b_hh, out_W, out_b, mem_keys, mem_values):
    raise NotImplementedError("write your pallas kernel here")


# speedup vs baseline: 4.1418x; 4.1418x over previous
import jax
import jax.numpy as jnp
from jax.experimental import pallas as pl
from jax.experimental.pallas import tpu as pltpu

B = 32
S = 20
K = 64
H = 128
SLOTS = 100000
BLK = 2000
NBLK = SLOTS // BLK
QROWS = (S - 1) * B


def _fused_kernel(inp_ref, trg_ref, h0_ref, c0_ref, embW_ref, embb_ref,
                  wih_ref, whh_ref, bias_ref, outW_ref, outb_ref,
                  keys_ref, vals_ref, out_ref, q_s, p_s, acc_s, l_s):
    i = pl.program_id(0)

    @pl.when(i == 0)
    def _prologue():
        emb = jnp.dot(inp_ref[:], embW_ref[:]) + embb_ref[:]
        xw = jnp.dot(emb, wih_ref[:]) + bias_ref[:]
        h = h0_ref[:]
        c = c0_ref[:]
        whh = whh_ref[:]
        outW = outW_ref[:]
        outb = outb_ref[:]
        for t in range(S):
            g = xw[t * B:(t + 1) * B, :] + jnp.dot(h, whh)
            ii = jax.nn.sigmoid(g[:, 0:H])
            ff = jax.nn.sigmoid(g[:, H:2 * H])
            gg = jnp.tanh(g[:, 2 * H:3 * H])
            oo = jax.nn.sigmoid(g[:, 3 * H:4 * H])
            c = ff * c + ii * gg
            h = oo * jnp.tanh(c)
            pred = jax.nn.sigmoid(jnp.dot(h, outW) + outb)
            p_s[t * B:(t + 1) * B, :] = pred
            if t < S - 1:
                err = trg_ref[t * B:(t + 1) * B, :] - pred
                qn = jnp.maximum(
                    jnp.sqrt(jnp.sum(err * err, axis=1, keepdims=True)), 1e-8)
                q_s[t * B:(t + 1) * B, :] = err / qn
        acc_s[:] = jnp.zeros_like(acc_s)
        l_s[:] = jnp.zeros_like(l_s)

    kb = keys_ref[:]
    kn = jnp.maximum(jnp.sqrt(jnp.sum(kb * kb, axis=1, keepdims=True)), 1e-8)
    kbn = kb / kn
    s = jax.lax.dot_general(q_s[:], kbn, (((1,), (1,)), ((), ())))
    p = jnp.exp(s)
    l_s[:] += jnp.sum(p, axis=1, keepdims=True)
    acc_s[:] += jnp.dot(p, vals_ref[:])

    @pl.when(i == NBLK - 1)
    def _epilogue():
        contents = acc_s[:] / l_s[:]
        out_ref[0:B, :] = p_s[0:B, :]
        out_ref[B:, :] = p_s[B:, :] + contents


def _const(shape):
    return pl.BlockSpec(shape, lambda i: (0,) * len(shape))


def _run(inp2, trg2, h0b, c0b, embWt, embb, wiht, whht, bias, outWt, outb,
         mem_keys, mem_values):
    return pl.pallas_call(
        _fused_kernel,
        grid=(NBLK,),
        in_specs=[
            _const((S * B, K)),
            _const((S * B, K)),
            _const((B, H)),
            _const((B, H)),
            _const((K, H)),
            _const((1, H)),
            _const((H, 4 * H)),
            _const((H, 4 * H)),
            _const((1, 4 * H)),
            _const((H, K)),
            _const((1, K)),
            pl.BlockSpec((BLK, K), lambda i: (i, 0)),
            pl.BlockSpec((BLK, K), lambda i: (i, 0)),
        ],
        out_specs=_const((S * B, K)),
        out_shape=jax.ShapeDtypeStruct((S * B, K), jnp.float32),
        scratch_shapes=[
            pltpu.VMEM((QROWS, K), jnp.float32),
            pltpu.VMEM((S * B, K), jnp.float32),
            pltpu.VMEM((QROWS, K), jnp.float32),
            pltpu.VMEM((QROWS, 1), jnp.float32),
        ],
        compiler_params=pltpu.CompilerParams(
            dimension_semantics=("arbitrary",)),
    )(inp2, trg2, h0b, c0b, embWt, embb, wiht, whht, bias, outWt, outb,
      mem_keys, mem_values)


def kernel(inp_seq, trg_seq, h0, c0, emb_W, emb_b, lstm_W_ih, lstm_W_hh,
           lstm_b_ih, lstm_b_hh, out_W, out_b, mem_keys, mem_values):
    inp2 = jnp.swapaxes(inp_seq, 0, 1).reshape(S * B, K)
    trg2 = jnp.swapaxes(trg_seq, 0, 1).reshape(S * B, K)
    out2 = _run(inp2, trg2, h0[0], c0[0], emb_W.T, emb_b.reshape(1, H),
                lstm_W_ih.T, lstm_W_hh.T,
                (lstm_b_ih + lstm_b_hh).reshape(1, 4 * H),
                out_W.T, out_b.reshape(1, K), mem_keys, mem_values)
    return out2.reshape(S, B, K).swapaxes(0, 1)


# speedup vs baseline: 4.4538x; 1.0753x over previous
import jax
import jax.numpy as jnp
from jax.experimental import pallas as pl
from jax.experimental.pallas import tpu as pltpu

B = 32
S = 20
K = 64
H = 128
SLOTS = 100000
BLK = 4000
NBLK = SLOTS // BLK
QROWS = (S - 1) * B


def _fused_kernel(inp_ref, trg_ref, h0_ref, c0_ref, embW_ref, embb_ref,
                  wih_ref, whh_ref, bias_ref, outW_ref, outb_ref,
                  keys_ref, vals_ref, out_ref, q_s, p_s, acc_s):
    i = pl.program_id(0)

    @pl.when(i == 0)
    def _prologue():
        emb = jnp.dot(inp_ref[:], embW_ref[:]) + embb_ref[:]
        xw = jnp.dot(emb, wih_ref[:]) + bias_ref[:]
        h = h0_ref[:]
        c = c0_ref[:]
        whh = whh_ref[:]
        outW = outW_ref[:]
        outb = outb_ref[:]
        for t in range(S):
            g = xw[t * B:(t + 1) * B, :] + jnp.dot(h, whh)
            ii = jax.nn.sigmoid(g[:, 0:H])
            ff = jax.nn.sigmoid(g[:, H:2 * H])
            gg = jnp.tanh(g[:, 2 * H:3 * H])
            oo = jax.nn.sigmoid(g[:, 3 * H:4 * H])
            c = ff * c + ii * gg
            h = oo * jnp.tanh(c)
            pred = jax.nn.sigmoid(jnp.dot(h, outW) + outb)
            p_s[t * B:(t + 1) * B, :] = pred
            if t < S - 1:
                err = trg_ref[t * B:(t + 1) * B, :] - pred
                qn = jnp.maximum(
                    jnp.sqrt(jnp.sum(err * err, axis=1, keepdims=True)), 1e-8)
                q_s[t * B:(t + 1) * B, :] = err / qn
        acc_s[:] = jnp.zeros_like(acc_s)

    kb = keys_ref[:]
    ksq = jnp.dot(kb * kb, jnp.ones((K, K), jnp.float32))
    kn = jnp.maximum(jnp.sqrt(ksq), 1e-8)
    kbn = kb / kn
    s = jax.lax.dot_general(q_s[:], kbn, (((1,), (1,)), ((), ())))
    p = jnp.exp(s)
    vb = jnp.concatenate(
        [vals_ref[:], jnp.ones((BLK, K), jnp.float32)], axis=1)
    acc_s[:] += jnp.dot(p, vb)

    @pl.when(i == NBLK - 1)
    def _epilogue():
        contents = acc_s[:, 0:K] / acc_s[:, K:K + 1]
        out_ref[0:B, :] = p_s[0:B, :]
        out_ref[B:, :] = p_s[B:, :] + contents


def _const(shape):
    return pl.BlockSpec(shape, lambda i: (0,) * len(shape))


def _run(inp2, trg2, h0b, c0b, embWt, embb, wiht, whht, bias, outWt, outb,
         mem_keys, mem_values):
    return pl.pallas_call(
        _fused_kernel,
        grid=(NBLK,),
        in_specs=[
            _const((S * B, K)),
            _const((S * B, K)),
            _const((B, H)),
            _const((B, H)),
            _const((K, H)),
            _const((1, H)),
            _const((H, 4 * H)),
            _const((H, 4 * H)),
            _const((1, 4 * H)),
            _const((H, K)),
            _const((1, K)),
            pl.BlockSpec((BLK, K), lambda i: (i, 0)),
            pl.BlockSpec((BLK, K), lambda i: (i, 0)),
        ],
        out_specs=_const((S * B, K)),
        out_shape=jax.ShapeDtypeStruct((S * B, K), jnp.float32),
        scratch_shapes=[
            pltpu.VMEM((QROWS, K), jnp.float32),
            pltpu.VMEM((S * B, K), jnp.float32),
            pltpu.VMEM((QROWS, 2 * K), jnp.float32),
        ],
        compiler_params=pltpu.CompilerParams(
            dimension_semantics=("arbitrary",)),
    )(inp2, trg2, h0b, c0b, embWt, embb, wiht, whht, bias, outWt, outb,
      mem_keys, mem_values)


def kernel(inp_seq, trg_seq, h0, c0, emb_W, emb_b, lstm_W_ih, lstm_W_hh,
           lstm_b_ih, lstm_b_hh, out_W, out_b, mem_keys, mem_values):
    inp2 = jnp.swapaxes(inp_seq, 0, 1).reshape(S * B, K)
    trg2 = jnp.swapaxes(trg_seq, 0, 1).reshape(S * B, K)
    out2 = _run(inp2, trg2, h0[0], c0[0], emb_W.T, emb_b.reshape(1, H),
                lstm_W_ih.T, lstm_W_hh.T,
                (lstm_b_ih + lstm_b_hh).reshape(1, 4 * H),
                out_W.T, out_b.reshape(1, K), mem_keys, mem_values)
    return out2.reshape(S, B, K).swapaxes(0, 1)
